# baseline (speedup 1.0000x reference)
import functools

import jax
import jax.numpy as jnp
from jax.experimental import pallas as pl
from jax.experimental.pallas import tpu as pltpu


def _se_step(x_ref, w1_ref, b1_ref, w2t_ref, b2_ref, o_ref, *, inv_hw):
    x = x_ref[...]
    s = jnp.sum(x, axis=(1, 2), dtype=jnp.float32) * jnp.float32(inv_hw)
    h = jax.lax.dot_general(s, w1_ref[...], (((1,), (1,)), ((), ())),
                            preferred_element_type=jnp.float32) + b1_ref[...]
    h = h * jax.nn.sigmoid(h)
    g = jnp.dot(h, w2t_ref[...], preferred_element_type=jnp.float32) + b2_ref[...]
    g = jax.nn.sigmoid(g)
    o_ref[...] = x * g[:, None, None, :]


def kernel(x, w1, b1, w2, b2):
    N, C, H, W = x.shape
    R = w1.shape[0]

    xt = jnp.transpose(x, (0, 2, 3, 1))
    w2t = jnp.transpose(w2, (1, 0))

    per_sample = C * H * W * jnp.dtype(x.dtype).itemsize
    nb = 1
    for d in range(1, N + 1):
        if N % d == 0 and N // d >= 8 and 4 * d * per_sample <= (48 << 20):
            nb = d

    out = pl.pallas_call(
        functools.partial(_se_step, inv_hw=1.0 / (H * W)),
        out_shape=jax.ShapeDtypeStruct((N, H, W, C), x.dtype),
        grid=(N // nb,),
        in_specs=[
            pl.BlockSpec((nb, H, W, C), lambda i: (i, 0, 0, 0)),
            pl.BlockSpec((R, C), lambda i: (0, 0)),
            pl.BlockSpec((1, R), lambda i: (0, 0)),
            pl.BlockSpec((R, C), lambda i: (0, 0)),
            pl.BlockSpec((1, C), lambda i: (0, 0)),
        ],
        out_specs=pl.BlockSpec((nb, H, W, C), lambda i: (i, 0, 0, 0)),
        compiler_params=pltpu.CompilerParams(
            dimension_semantics=("parallel",),
            vmem_limit_bytes=int(56 << 20)),
    )(xt, w1, b1.reshape(1, R), w2t, b2.reshape(1, C))

    return jnp.transpose(out, (0, 3, 1, 2))


# speedup vs baseline: 1.0721x; 1.0721x over previous
import functools

import jax
import jax.numpy as jnp
from jax.experimental import pallas as pl
from jax.experimental.pallas import tpu as pltpu


def _se_step(x_ref, w1_ref, b1_ref, w2t_ref, b2_ref, o_ref, *, inv_hw):
    x = x_ref[...]
    s = jnp.sum(x, axis=(1, 2), dtype=jnp.float32) * jnp.float32(inv_hw)
    h = jax.lax.dot_general(s, w1_ref[...], (((1,), (1,)), ((), ())),
                            preferred_element_type=jnp.float32) + b1_ref[...]
    h = h * jax.nn.sigmoid(h)
    g = jnp.dot(h, w2t_ref[...], preferred_element_type=jnp.float32) + b2_ref[...]
    g = jax.nn.sigmoid(g)
    o_ref[...] = x * g[:, None, None, :]


def kernel(x, w1, b1, w2, b2):
    N, C, H, W = x.shape
    R = w1.shape[0]

    xt = jnp.transpose(x, (0, 2, 3, 1))
    w2t = jnp.transpose(w2, (1, 0))

    per_sample = C * H * W * jnp.dtype(x.dtype).itemsize
    nb = 1
    for d in range(1, N + 1):
        if N % d == 0 and N // d >= 4 and 4 * d * per_sample <= (48 << 20):
            nb = d

    out = pl.pallas_call(
        functools.partial(_se_step, inv_hw=1.0 / (H * W)),
        out_shape=jax.ShapeDtypeStruct((N, H, W, C), x.dtype),
        grid=(N // nb,),
        in_specs=[
            pl.BlockSpec((nb, H, W, C), lambda i: (i, 0, 0, 0)),
            pl.BlockSpec((R, C), lambda i: (0, 0)),
            pl.BlockSpec((1, R), lambda i: (0, 0)),
            pl.BlockSpec((R, C), lambda i: (0, 0)),
            pl.BlockSpec((1, C), lambda i: (0, 0)),
        ],
        out_specs=pl.BlockSpec((nb, H, W, C), lambda i: (i, 0, 0, 0)),
        compiler_params=pltpu.CompilerParams(
            dimension_semantics=("parallel",),
            vmem_limit_bytes=int(56 << 20)),
    )(xt, w1, b1.reshape(1, R), w2t, b2.reshape(1, C))

    return jnp.transpose(out, (0, 3, 1, 2))
